# Initial kernel scaffold; baseline (speedup 1.0000x reference)
import jax
import jax.numpy as jnp
from jax.experimental import pallas as pl


def kernel(X, M0, W_q, b_q, W_k, b_k, W_v, b_v, W_g, b_g):
    raise NotImplementedError("write your pallas kernel here")


# speedup vs baseline: 273.5257x; 273.5257x over previous
import functools

import jax
import jax.numpy as jnp
from jax.experimental import pallas as pl
from jax.experimental.pallas import tpu as pltpu

L = 2048
B = 2
D = 768
d = 128
NM = 8
TOPK = 2
CHUNK = 256
F32 = jnp.float32


def _mom_kernel(x_ref, m0_ref, wq_ref, bq_ref, wk_ref, bk_ref, wv_ref,
                bv_ref, wg_ref, bg_ref, out_ref, q_scr, w_scr):
    m = pl.program_id(1)

    @pl.when(m == 0)
    def _setup():
        x_full = x_ref[0]
        g = jax.lax.dot_general(x_full, wg_ref[...],
                                (((1,), (1,)), ((), ())),
                                preferred_element_type=F32) + bg_ref[...]
        gmax = jnp.max(g, axis=1, keepdims=True)
        e = jnp.exp(g - gmax)
        s = e / jnp.sum(e, axis=1, keepdims=True)
        iota = jax.lax.broadcasted_iota(jnp.int32, (L, NM), 1)
        v1 = jnp.max(s, axis=1, keepdims=True)
        i1 = jnp.min(jnp.where(s == v1, iota, NM), axis=1, keepdims=True)
        s2 = jnp.where(iota == i1, -jnp.inf, s)
        v2 = jnp.max(s2, axis=1, keepdims=True)
        i2 = jnp.min(jnp.where(s2 == v2, iota, NM), axis=1, keepdims=True)
        sel = (iota == i1) | (iota == i2)
        alpha = s / (v1 + v2)
        w8 = jnp.where(sel, alpha, -1.0)
        pad = jnp.full((L, 16 - 1 - NM), -1.0, dtype=F32)
        w_scr[...] = jnp.concatenate(
            [jnp.ones((L, 1), dtype=F32), w8, pad], axis=1)
        q_scr[...] = jax.lax.dot_general(
            x_full, wq_ref[...], (((1,), (1,)), ((), ())),
            preferred_element_type=F32) + bq_ref[...]

    wk_m = wk_ref[pl.ds(m * d, d), :]
    wv_m = wv_ref[pl.ds(m * d, d), :]
    bk_m = bk_ref[pl.ds(m, 1), :]
    bv_m = bv_ref[pl.ds(m, 1), :]
    m0 = m0_ref[...]

    row_i = jax.lax.broadcasted_iota(jnp.int32, (CHUNK, CHUNK), 0)
    col_i = jax.lax.broadcasted_iota(jnp.int32, (CHUNK, CHUNK), 1)
    causal = row_i >= col_i

    def chunk_body(i, carry):
        t0 = i * CHUNK
        x = x_ref[0, pl.ds(t0, CHUNK), :]
        q = q_scr[pl.ds(t0, CHUNK), :]
        w16 = w_scr[pl.ds(t0, CHUNK), :]
        lane = jax.lax.broadcasted_iota(jnp.int32, (CHUNK, 16), 1)
        wrow = jnp.sum(jnp.where(lane == m, w16, 0.0), axis=1,
                       keepdims=True)
        selc = wrow >= 0.0
        weight = jnp.maximum(wrow, 0.0)
        k = jax.lax.dot_general(x, wk_m, (((1,), (1,)), ((), ())),
                                preferred_element_type=F32) + bk_m
        k = jnp.where(selc, k, 0.0)
        v = jax.lax.dot_general(x, wv_m, (((1,), (1,)), ((), ())),
                                preferred_element_type=F32) + bv_m
        a = jax.lax.dot_general(q, k, (((1,), (1,)), ((), ())),
                                preferred_element_type=F32)
        a = jnp.where(causal, a, 0.0)
        o = (jnp.dot(q, carry, preferred_element_type=F32)
             + jnp.dot(a, v, preferred_element_type=F32))
        contrib = weight * o

        @pl.when(m == 0)
        def _init():
            out_ref[0, pl.ds(t0, CHUNK), :] = contrib

        @pl.when(m > 0)
        def _acc():
            out_ref[0, pl.ds(t0, CHUNK), :] += contrib

        return carry + jax.lax.dot_general(
            k, v, (((0,), (0,)), ((), ())), preferred_element_type=F32)

    jax.lax.fori_loop(0, L // CHUNK, chunk_body, m0)


@functools.partial(jax.jit, static_argnames=("interpret",))
def kernel(X, M0, W_q, b_q, W_k, b_k, W_v, b_v, W_g, b_g, interpret=False):
    Mp1 = NM + 1
    xb = jnp.transpose(X, (1, 0, 2))
    bq2 = b_q.reshape(1, d)
    bk2 = b_k.reshape(Mp1, d)
    bv2 = b_v.reshape(Mp1, d)
    bg2 = b_g.reshape(1, NM)

    out = pl.pallas_call(
        _mom_kernel,
        grid=(B, Mp1),
        in_specs=[
            pl.BlockSpec((1, L, D), lambda b, m: (b, 0, 0)),
            pl.BlockSpec((d, d), lambda b, m: (0, 0)),
            pl.BlockSpec((d, D), lambda b, m: (0, 0)),
            pl.BlockSpec((1, d), lambda b, m: (0, 0)),
            pl.BlockSpec((d * Mp1, D), lambda b, m: (0, 0)),
            pl.BlockSpec((Mp1, d), lambda b, m: (0, 0)),
            pl.BlockSpec((d * Mp1, D), lambda b, m: (0, 0)),
            pl.BlockSpec((Mp1, d), lambda b, m: (0, 0)),
            pl.BlockSpec((NM, D), lambda b, m: (0, 0)),
            pl.BlockSpec((1, NM), lambda b, m: (0, 0)),
        ],
        out_specs=pl.BlockSpec((1, L, d), lambda b, m: (b, 0, 0)),
        out_shape=jax.ShapeDtypeStruct((B, L, d), F32),
        scratch_shapes=[
            pltpu.VMEM((L, d), F32),
            pltpu.VMEM((L, 16), F32),
        ],
        compiler_params=pltpu.CompilerParams(
            dimension_semantics=("arbitrary", "arbitrary"),
        ),
        interpret=interpret,
    )(xb, M0, W_q, bq2, W_k, bk2, W_v, bv2, W_g, bg2)

    return jnp.transpose(out, (1, 0, 2))


# speedup vs baseline: 278.0332x; 1.0165x over previous
import functools

import jax
import jax.numpy as jnp
from jax.experimental import pallas as pl
from jax.experimental.pallas import tpu as pltpu

L = 2048
B = 2
D = 768
d = 128
NM = 8
TOPK = 2
CHUNK = 256
F32 = jnp.float32


def _mom_kernel(x_ref, m0_ref, wq_ref, bq_ref, wk_ref, bk_ref, wv_ref,
                bv_ref, wg_ref, bg_ref, out_ref, q_scr, w_scr):
    b = pl.program_id(0)
    m = pl.program_id(1)

    @pl.when(m == 0)
    def _setup():
        xb = x_ref[:, pl.ds(b * D, D)]
        gt = jax.lax.dot_general(wg_ref[...], xb,
                                 (((1,), (1,)), ((), ())),
                                 preferred_element_type=F32) + bg_ref[...]
        gmax = jnp.max(gt, axis=0, keepdims=True)
        e = jnp.exp(gt - gmax)
        s = e / jnp.sum(e, axis=0, keepdims=True)
        v1 = jnp.max(s, axis=0, keepdims=True)
        c1 = jnp.sum(jnp.where(s == v1, 1.0, 0.0), axis=0, keepdims=True)
        m2 = jnp.max(jnp.where(s == v1, -jnp.inf, s), axis=0, keepdims=True)
        v2 = jnp.where(c1 >= 2.0, v1, m2)
        sel = s >= v2
        alpha = s / (v1 + v2)
        w8t = jnp.where(sel, alpha, -1.0)
        w16t = jnp.concatenate(
            [jnp.ones((1, L), dtype=F32), w8t,
             jnp.full((16 - 1 - NM, L), -1.0, dtype=F32)], axis=0)
        w_scr[...] = w16t.T
        q_scr[...] = jax.lax.dot_general(
            xb, wq_ref[...], (((1,), (1,)), ((), ())),
            preferred_element_type=F32) + bq_ref[...]

    wk_m = wk_ref[pl.ds(m * d, d), :]
    wv_m = wv_ref[pl.ds(m * d, d), :]
    bk_m = bk_ref[pl.ds(m, 1), :]
    bv_m = bv_ref[pl.ds(m, 1), :]
    m0 = m0_ref[...]

    row_i = jax.lax.broadcasted_iota(jnp.int32, (CHUNK, CHUNK), 0)
    col_i = jax.lax.broadcasted_iota(jnp.int32, (CHUNK, CHUNK), 1)
    causal = row_i >= col_i

    def chunk_body(i, carry):
        t0 = i * CHUNK
        x = x_ref[pl.ds(t0, CHUNK), pl.ds(b * D, D)]
        q = q_scr[pl.ds(t0, CHUNK), :]
        w16 = w_scr[pl.ds(t0, CHUNK), :]
        lane = jax.lax.broadcasted_iota(jnp.int32, (CHUNK, 16), 1)
        wrow = jnp.sum(jnp.where(lane == m, w16, 0.0), axis=1,
                       keepdims=True)
        selc = wrow >= 0.0
        weight = jnp.maximum(wrow, 0.0)
        k = jax.lax.dot_general(x, wk_m, (((1,), (1,)), ((), ())),
                                preferred_element_type=F32) + bk_m
        k = jnp.where(selc, k, 0.0)
        v = jax.lax.dot_general(x, wv_m, (((1,), (1,)), ((), ())),
                                preferred_element_type=F32) + bv_m
        a = jax.lax.dot_general(q, k, (((1,), (1,)), ((), ())),
                                preferred_element_type=F32)
        a = jnp.where(causal, a, 0.0)
        o = (jnp.dot(q, carry, preferred_element_type=F32)
             + jnp.dot(a, v, preferred_element_type=F32))
        contrib = weight * o

        @pl.when(m == 0)
        def _init():
            out_ref[pl.ds(t0, CHUNK), pl.ds(b * d, d)] = contrib

        @pl.when(m > 0)
        def _acc():
            out_ref[pl.ds(t0, CHUNK), pl.ds(b * d, d)] += contrib

        return carry + jax.lax.dot_general(
            k, v, (((0,), (0,)), ((), ())), preferred_element_type=F32)

    jax.lax.fori_loop(0, L // CHUNK, chunk_body, m0)


@functools.partial(jax.jit, static_argnames=("interpret",))
def kernel(X, M0, W_q, b_q, W_k, b_k, W_v, b_v, W_g, b_g, interpret=False):
    Mp1 = NM + 1
    x2 = X.reshape(L, B * D)
    bq2 = b_q.reshape(1, d)
    bk2 = b_k.reshape(Mp1, d)
    bv2 = b_v.reshape(Mp1, d)
    bg2 = b_g.reshape(NM, 1)

    out = pl.pallas_call(
        _mom_kernel,
        grid=(B, Mp1),
        in_specs=[
            pl.BlockSpec((L, B * D), lambda b, m: (0, 0)),
            pl.BlockSpec((d, d), lambda b, m: (0, 0)),
            pl.BlockSpec((d, D), lambda b, m: (0, 0)),
            pl.BlockSpec((1, d), lambda b, m: (0, 0)),
            pl.BlockSpec((d * Mp1, D), lambda b, m: (0, 0)),
            pl.BlockSpec((Mp1, d), lambda b, m: (0, 0)),
            pl.BlockSpec((d * Mp1, D), lambda b, m: (0, 0)),
            pl.BlockSpec((Mp1, d), lambda b, m: (0, 0)),
            pl.BlockSpec((NM, D), lambda b, m: (0, 0)),
            pl.BlockSpec((NM, 1), lambda b, m: (0, 0)),
        ],
        out_specs=pl.BlockSpec((L, B * d), lambda b, m: (0, 0)),
        out_shape=jax.ShapeDtypeStruct((L, B * d), F32),
        scratch_shapes=[
            pltpu.VMEM((L, d), F32),
            pltpu.VMEM((L, 16), F32),
        ],
        compiler_params=pltpu.CompilerParams(
            dimension_semantics=("arbitrary", "arbitrary"),
        ),
        interpret=interpret,
    )(x2, M0, W_q, bq2, W_k, bk2, W_v, bv2, W_g, bg2)

    return out.reshape(L, B, d)


# speedup vs baseline: 279.5721x; 1.0055x over previous
import functools

import jax
import jax.numpy as jnp
from jax.experimental import pallas as pl
from jax.experimental.pallas import tpu as pltpu

L = 2048
B = 2
D = 768
d = 128
NM = 8
TOPK = 2
CHUNK = 256
F32 = jnp.float32


def _mom_kernel(x_ref, m0_ref, wq_ref, bq_ref, wk_ref, bk_ref, wv_ref,
                bv_ref, wg_ref, bg_ref, out_ref, q_scr, w_scr, x_scr):
    b = pl.program_id(0)
    m = pl.program_id(1)

    @pl.when(m == 0)
    def _setup():
        xb = x_ref[:, pl.ds(b * D, D)]
        x_scr[...] = xb.astype(jnp.bfloat16)
        gt = jax.lax.dot_general(wg_ref[...], xb,
                                 (((1,), (1,)), ((), ())),
                                 preferred_element_type=F32) + bg_ref[...]
        gmax = jnp.max(gt, axis=0, keepdims=True)
        e = jnp.exp(gt - gmax)
        s = e / jnp.sum(e, axis=0, keepdims=True)
        v1 = jnp.max(s, axis=0, keepdims=True)
        c1 = jnp.sum(jnp.where(s == v1, 1.0, 0.0), axis=0, keepdims=True)
        m2 = jnp.max(jnp.where(s == v1, -jnp.inf, s), axis=0, keepdims=True)
        v2 = jnp.where(c1 >= 2.0, v1, m2)
        sel = s >= v2
        alpha = s / (v1 + v2)
        w8t = jnp.where(sel, alpha, -1.0)
        w16t = jnp.concatenate(
            [jnp.ones((1, L), dtype=F32), w8t,
             jnp.full((16 - 1 - NM, L), -1.0, dtype=F32)], axis=0)
        w_scr[...] = w16t.T
        q_scr[...] = jax.lax.dot_general(
            xb, wq_ref[...], (((1,), (1,)), ((), ())),
            preferred_element_type=F32) + bq_ref[...]

    wk_m = wk_ref[pl.ds(m * d, d), :]
    wv_m = wv_ref[pl.ds(m * d, d), :]
    bk_m = bk_ref[pl.ds(m, 1), :]
    bv_m = bv_ref[pl.ds(m, 1), :]
    m0 = m0_ref[...]

    row_i = jax.lax.broadcasted_iota(jnp.int32, (CHUNK, CHUNK), 0)
    col_i = jax.lax.broadcasted_iota(jnp.int32, (CHUNK, CHUNK), 1)
    causal = row_i >= col_i

    def chunk_body(i, carry):
        t0 = i * CHUNK
        x = x_scr[pl.ds(t0, CHUNK), :]
        q = q_scr[pl.ds(t0, CHUNK), :]
        w16 = w_scr[pl.ds(t0, CHUNK), :]
        lane = jax.lax.broadcasted_iota(jnp.int32, (CHUNK, 16), 1)
        wrow = jnp.sum(jnp.where(lane == m, w16, 0.0), axis=1,
                       keepdims=True)
        selc = wrow >= 0.0
        weight = jnp.maximum(wrow, 0.0)
        k = jax.lax.dot_general(x, wk_m, (((1,), (1,)), ((), ())),
                                preferred_element_type=F32) + bk_m
        k = jnp.where(selc, k, 0.0)
        v = jax.lax.dot_general(x, wv_m, (((1,), (1,)), ((), ())),
                                preferred_element_type=F32) + bv_m
        a = jax.lax.dot_general(q, k, (((1,), (1,)), ((), ())),
                                preferred_element_type=F32)
        a = jnp.where(causal, a, 0.0)
        o = (jnp.dot(q, carry, preferred_element_type=F32)
             + jnp.dot(a, v, preferred_element_type=F32))
        contrib = weight * o

        @pl.when(m == 0)
        def _init():
            out_ref[pl.ds(t0, CHUNK), pl.ds(b * d, d)] = contrib

        @pl.when(m > 0)
        def _acc():
            out_ref[pl.ds(t0, CHUNK), pl.ds(b * d, d)] += contrib

        return carry + jax.lax.dot_general(
            k, v, (((0,), (0,)), ((), ())), preferred_element_type=F32)

    jax.lax.fori_loop(0, L // CHUNK, chunk_body, m0)


@functools.partial(jax.jit, static_argnames=("interpret",))
def kernel(X, M0, W_q, b_q, W_k, b_k, W_v, b_v, W_g, b_g, interpret=False):
    Mp1 = NM + 1
    x2 = X.reshape(L, B * D)
    wk_bf = W_k.astype(jnp.bfloat16)
    wv_bf = W_v.astype(jnp.bfloat16)
    bq2 = b_q.reshape(1, d)
    bk2 = b_k.reshape(Mp1, d)
    bv2 = b_v.reshape(Mp1, d)
    bg2 = b_g.reshape(NM, 1)

    out = pl.pallas_call(
        _mom_kernel,
        grid=(B, Mp1),
        in_specs=[
            pl.BlockSpec((L, B * D), lambda b, m: (0, 0)),
            pl.BlockSpec((d, d), lambda b, m: (0, 0)),
            pl.BlockSpec((d, D), lambda b, m: (0, 0)),
            pl.BlockSpec((1, d), lambda b, m: (0, 0)),
            pl.BlockSpec((d * Mp1, D), lambda b, m: (0, 0)),
            pl.BlockSpec((Mp1, d), lambda b, m: (0, 0)),
            pl.BlockSpec((d * Mp1, D), lambda b, m: (0, 0)),
            pl.BlockSpec((Mp1, d), lambda b, m: (0, 0)),
            pl.BlockSpec((NM, D), lambda b, m: (0, 0)),
            pl.BlockSpec((NM, 1), lambda b, m: (0, 0)),
        ],
        out_specs=pl.BlockSpec((L, B * d), lambda b, m: (0, 0)),
        out_shape=jax.ShapeDtypeStruct((L, B * d), F32),
        scratch_shapes=[
            pltpu.VMEM((L, d), F32),
            pltpu.VMEM((L, 16), F32),
            pltpu.VMEM((L, D), jnp.bfloat16),
        ],
        compiler_params=pltpu.CompilerParams(
            dimension_semantics=("arbitrary", "arbitrary"),
        ),
        interpret=interpret,
    )(x2, M0, W_q, bq2, wk_bf, bk2, wv_bf, bv2, W_g, bg2)

    return out.reshape(L, B, d)


# speedup vs baseline: 281.9093x; 1.0084x over previous
import functools

import jax
import jax.numpy as jnp
from jax.experimental import pallas as pl
from jax.experimental.pallas import tpu as pltpu

L = 2048
B = 2
D = 768
d = 128
NM = 8
TOPK = 2
CHUNK = 256
F32 = jnp.float32


def _mom_kernel(x_ref, m0_ref, wq_ref, bq_ref, wk_ref, bk_ref, wv_ref,
                bv_ref, wg_ref, bg_ref, out_ref, q_scr, w_scr, x_scr):
    b = pl.program_id(0)

    def _setup():
        xb = x_ref[:, pl.ds(b * D, D)]
        x_scr[...] = xb.astype(jnp.bfloat16)
        gt = jax.lax.dot_general(wg_ref[...], xb,
                                 (((1,), (1,)), ((), ())),
                                 preferred_element_type=F32) + bg_ref[...]
        gmax = jnp.max(gt, axis=0, keepdims=True)
        e = jnp.exp(gt - gmax)
        s = e / jnp.sum(e, axis=0, keepdims=True)
        v1 = jnp.max(s, axis=0, keepdims=True)
        c1 = jnp.sum(jnp.where(s == v1, 1.0, 0.0), axis=0, keepdims=True)
        m2 = jnp.max(jnp.where(s == v1, -jnp.inf, s), axis=0, keepdims=True)
        v2 = jnp.where(c1 >= 2.0, v1, m2)
        sel = s >= v2
        alpha = s / (v1 + v2)
        w8t = jnp.where(sel, alpha, -1.0)
        w16t = jnp.concatenate(
            [jnp.ones((1, L), dtype=F32), w8t,
             jnp.full((16 - 1 - NM, L), -1.0, dtype=F32)], axis=0)
        w_scr[...] = w16t.T
        q_scr[...] = jax.lax.dot_general(
            xb, wq_ref[...], (((1,), (1,)), ((), ())),
            preferred_element_type=F32) + bq_ref[...]

    _setup()

    m0 = m0_ref[...]
    row_i = jax.lax.broadcasted_iota(jnp.int32, (CHUNK, CHUNK), 0)
    col_i = jax.lax.broadcasted_iota(jnp.int32, (CHUNK, CHUNK), 1)
    causal = row_i >= col_i

    def mem_body(m, _):
        wk_m = wk_ref[pl.ds(m * d, d), :]
        wv_m = wv_ref[pl.ds(m * d, d), :]
        bk_m = bk_ref[pl.ds(m, 1), :]
        bv_m = bv_ref[pl.ds(m, 1), :]

        def chunk_body(i, carry):
            t0 = i * CHUNK
            x = x_scr[pl.ds(t0, CHUNK), :]
            q = q_scr[pl.ds(t0, CHUNK), :]
            w16 = w_scr[pl.ds(t0, CHUNK), :]
            lane = jax.lax.broadcasted_iota(jnp.int32, (CHUNK, 16), 1)
            wrow = jnp.sum(jnp.where(lane == m, w16, 0.0), axis=1,
                           keepdims=True)
            selc = wrow >= 0.0
            weight = jnp.maximum(wrow, 0.0)
            k = jax.lax.dot_general(x, wk_m, (((1,), (1,)), ((), ())),
                                    preferred_element_type=F32) + bk_m
            k = jnp.where(selc, k, 0.0)
            v = jax.lax.dot_general(x, wv_m, (((1,), (1,)), ((), ())),
                                    preferred_element_type=F32) + bv_m
            a = jax.lax.dot_general(q, k, (((1,), (1,)), ((), ())),
                                    preferred_element_type=F32)
            a = jnp.where(causal, a, 0.0)
            o = (jnp.dot(q, carry, preferred_element_type=F32)
                 + jnp.dot(a, v, preferred_element_type=F32))
            contrib = weight * o

            @pl.when(m == 0)
            def _init():
                out_ref[pl.ds(t0, CHUNK), pl.ds(b * d, d)] = contrib

            @pl.when(m > 0)
            def _acc():
                out_ref[pl.ds(t0, CHUNK), pl.ds(b * d, d)] += contrib

            return carry + jax.lax.dot_general(
                k, v, (((0,), (0,)), ((), ())), preferred_element_type=F32)

        jax.lax.fori_loop(0, L // CHUNK, chunk_body, m0)
        return 0

    jax.lax.fori_loop(0, NM + 1, mem_body, 0)


@functools.partial(jax.jit, static_argnames=("interpret",))
def kernel(X, M0, W_q, b_q, W_k, b_k, W_v, b_v, W_g, b_g, interpret=False):
    Mp1 = NM + 1
    x2 = X.reshape(L, B * D)
    wk_bf = W_k.astype(jnp.bfloat16)
    wv_bf = W_v.astype(jnp.bfloat16)
    bq2 = b_q.reshape(1, d)
    bk2 = b_k.reshape(Mp1, d)
    bv2 = b_v.reshape(Mp1, d)
    bg2 = b_g.reshape(NM, 1)

    out = pl.pallas_call(
        _mom_kernel,
        grid=(B,),
        in_specs=[
            pl.BlockSpec((L, B * D), lambda b: (0, 0)),
            pl.BlockSpec((d, d), lambda b: (0, 0)),
            pl.BlockSpec((d, D), lambda b: (0, 0)),
            pl.BlockSpec((1, d), lambda b: (0, 0)),
            pl.BlockSpec((d * Mp1, D), lambda b: (0, 0)),
            pl.BlockSpec((Mp1, d), lambda b: (0, 0)),
            pl.BlockSpec((d * Mp1, D), lambda b: (0, 0)),
            pl.BlockSpec((Mp1, d), lambda b: (0, 0)),
            pl.BlockSpec((NM, D), lambda b: (0, 0)),
            pl.BlockSpec((NM, 1), lambda b: (0, 0)),
        ],
        out_specs=pl.BlockSpec((L, B * d), lambda b: (0, 0)),
        out_shape=jax.ShapeDtypeStruct((L, B * d), F32),
        scratch_shapes=[
            pltpu.VMEM((L, d), F32),
            pltpu.VMEM((L, 16), F32),
            pltpu.VMEM((L, D), jnp.bfloat16),
        ],
        compiler_params=pltpu.CompilerParams(
            dimension_semantics=("arbitrary",),
        ),
        interpret=interpret,
    )(x2, M0, W_q, bq2, wk_bf, bk2, wv_bf, bv2, W_g, bg2)

    return out.reshape(L, B, d)


# speedup vs baseline: 284.3100x; 1.0085x over previous
import functools

import jax
import jax.numpy as jnp
from jax.experimental import pallas as pl
from jax.experimental.pallas import tpu as pltpu

L = 2048
B = 2
D = 768
d = 128
NM = 8
TOPK = 2
CHUNK = 256
F32 = jnp.float32


def _mom_kernel(x_ref, m0_ref, wq_ref, bq_ref, wk_ref, bk_ref, wv_ref,
                bv_ref, wg_ref, bg_ref, out_ref, q_scr, w_scr, x_scr):
    b = pl.program_id(0)

    def _setup():
        xb = x_ref[...]
        x_scr[...] = xb.astype(jnp.bfloat16)
        gt = jax.lax.dot_general(wg_ref[...], xb,
                                 (((1,), (1,)), ((), ())),
                                 preferred_element_type=F32) + bg_ref[...]
        gmax = jnp.max(gt, axis=0, keepdims=True)
        e = jnp.exp(gt - gmax)
        s = e / jnp.sum(e, axis=0, keepdims=True)
        v1 = jnp.max(s, axis=0, keepdims=True)
        c1 = jnp.sum(jnp.where(s == v1, 1.0, 0.0), axis=0, keepdims=True)
        m2 = jnp.max(jnp.where(s == v1, -jnp.inf, s), axis=0, keepdims=True)
        v2 = jnp.where(c1 >= 2.0, v1, m2)
        sel = s >= v2
        alpha = s / (v1 + v2)
        w8t = jnp.where(sel, alpha, -1.0)
        w16t = jnp.concatenate(
            [jnp.ones((1, L), dtype=F32), w8t,
             jnp.full((16 - 1 - NM, L), -1.0, dtype=F32)], axis=0)
        w_scr[...] = w16t.T
        q_scr[...] = jax.lax.dot_general(
            xb, wq_ref[...], (((1,), (1,)), ((), ())),
            preferred_element_type=F32) + bq_ref[...]

    _setup()

    m0 = m0_ref[...]
    row_i = jax.lax.broadcasted_iota(jnp.int32, (CHUNK, CHUNK), 0)
    col_i = jax.lax.broadcasted_iota(jnp.int32, (CHUNK, CHUNK), 1)
    causal = row_i >= col_i

    def mem_body(m, _):
        wk_m = wk_ref[pl.ds(m * d, d), :]
        wv_m = wv_ref[pl.ds(m * d, d), :]
        bk_m = bk_ref[pl.ds(m, 1), :]
        bv_m = bv_ref[pl.ds(m, 1), :]

        def chunk_body(i, carry):
            t0 = i * CHUNK
            x = x_scr[pl.ds(t0, CHUNK), :]
            q = q_scr[pl.ds(t0, CHUNK), :]
            w16 = w_scr[pl.ds(t0, CHUNK), :]
            lane = jax.lax.broadcasted_iota(jnp.int32, (CHUNK, 16), 1)
            wrow = jnp.sum(jnp.where(lane == m, w16, 0.0), axis=1,
                           keepdims=True)
            selc = wrow >= 0.0
            weight = jnp.maximum(wrow, 0.0)
            k = jax.lax.dot_general(x, wk_m, (((1,), (1,)), ((), ())),
                                    preferred_element_type=F32) + bk_m
            k = jnp.where(selc, k, 0.0)
            v = jax.lax.dot_general(x, wv_m, (((1,), (1,)), ((), ())),
                                    preferred_element_type=F32) + bv_m
            a = jax.lax.dot_general(q, k, (((1,), (1,)), ((), ())),
                                    preferred_element_type=F32)
            a = jnp.where(causal, a, 0.0)
            o = (jnp.dot(q, carry, preferred_element_type=F32)
                 + jnp.dot(a, v, preferred_element_type=F32))
            contrib = weight * o

            @pl.when(m == 0)
            def _init():
                out_ref[pl.ds(t0, CHUNK), :] = contrib

            @pl.when(m > 0)
            def _acc():
                out_ref[pl.ds(t0, CHUNK), :] += contrib

            return carry + jax.lax.dot_general(
                k, v, (((0,), (0,)), ((), ())), preferred_element_type=F32)

        jax.lax.fori_loop(0, L // CHUNK, chunk_body, m0)
        return 0

    jax.lax.fori_loop(0, NM + 1, mem_body, 0)


@functools.partial(jax.jit, static_argnames=("interpret",))
def kernel(X, M0, W_q, b_q, W_k, b_k, W_v, b_v, W_g, b_g, interpret=False):
    Mp1 = NM + 1
    x2 = X.reshape(L, B * D)
    wk_bf = W_k.astype(jnp.bfloat16)
    wv_bf = W_v.astype(jnp.bfloat16)
    bq2 = b_q.reshape(1, d)
    bk2 = b_k.reshape(Mp1, d)
    bv2 = b_v.reshape(Mp1, d)
    bg2 = b_g.reshape(NM, 1)

    out = pl.pallas_call(
        _mom_kernel,
        grid=(B,),
        in_specs=[
            pl.BlockSpec((L, D), lambda b: (0, b)),
            pl.BlockSpec((d, d), lambda b: (0, 0)),
            pl.BlockSpec((d, D), lambda b: (0, 0)),
            pl.BlockSpec((1, d), lambda b: (0, 0)),
            pl.BlockSpec((d * Mp1, D), lambda b: (0, 0)),
            pl.BlockSpec((Mp1, d), lambda b: (0, 0)),
            pl.BlockSpec((d * Mp1, D), lambda b: (0, 0)),
            pl.BlockSpec((Mp1, d), lambda b: (0, 0)),
            pl.BlockSpec((NM, D), lambda b: (0, 0)),
            pl.BlockSpec((NM, 1), lambda b: (0, 0)),
        ],
        out_specs=pl.BlockSpec((L, d), lambda b: (0, b)),
        out_shape=jax.ShapeDtypeStruct((L, B * d), F32),
        scratch_shapes=[
            pltpu.VMEM((L, d), F32),
            pltpu.VMEM((L, 16), F32),
            pltpu.VMEM((L, D), jnp.bfloat16),
        ],
        compiler_params=pltpu.CompilerParams(
            dimension_semantics=("parallel",),
        ),
        interpret=interpret,
    )(x2, M0, W_q, bq2, wk_bf, bk2, wv_bf, bv2, W_g, bg2)

    return out.reshape(L, B, d)


# speedup vs baseline: 337.3545x; 1.1866x over previous
import functools

import jax
import jax.numpy as jnp
from jax.experimental import pallas as pl
from jax.experimental.pallas import tpu as pltpu

L = 2048
B = 2
D = 768
d = 128
NM = 8
TOPK = 2
CHUNK = 256
F32 = jnp.float32


def _mom_kernel(x_ref, m0_ref, wq_ref, bq_ref, wk_ref, bk_ref, wv_ref,
                bv_ref, wg_ref, bg_ref, out_ref, q_scr, w_scr, k_scr, v_scr):
    b = pl.program_id(0)

    def _setup():
        xb = x_ref[...]
        x16 = xb.astype(jnp.bfloat16)
        gt = jax.lax.dot_general(wg_ref[...], xb,
                                 (((1,), (1,)), ((), ())),
                                 preferred_element_type=F32) + bg_ref[...]
        gmax = jnp.max(gt, axis=0, keepdims=True)
        e = jnp.exp(gt - gmax)
        s = e / jnp.sum(e, axis=0, keepdims=True)
        v1 = jnp.max(s, axis=0, keepdims=True)
        c1 = jnp.sum(jnp.where(s == v1, 1.0, 0.0), axis=0, keepdims=True)
        m2 = jnp.max(jnp.where(s == v1, -jnp.inf, s), axis=0, keepdims=True)
        v2 = jnp.where(c1 >= 2.0, v1, m2)
        sel = s >= v2
        alpha = s / (v1 + v2)
        w8t = jnp.where(sel, alpha, -1.0)
        w16t = jnp.concatenate(
            [jnp.ones((1, L), dtype=F32), w8t,
             jnp.full((16 - 1 - NM, L), -1.0, dtype=F32)], axis=0)
        w_scr[...] = w16t.T
        q_scr[...] = jax.lax.dot_general(
            xb, wq_ref[...], (((1,), (1,)), ((), ())),
            preferred_element_type=F32) + bq_ref[...]
        k_scr[...] = jax.lax.dot_general(
            x16, wk_ref[...], (((1,), (1,)), ((), ())),
            preferred_element_type=F32) + bk_ref[...]
        v_scr[...] = jax.lax.dot_general(
            x16, wv_ref[...], (((1,), (1,)), ((), ())),
            preferred_element_type=F32) + bv_ref[...]

    _setup()

    m0 = m0_ref[...]
    row_i = jax.lax.broadcasted_iota(jnp.int32, (CHUNK, CHUNK), 0)
    col_i = jax.lax.broadcasted_iota(jnp.int32, (CHUNK, CHUNK), 1)
    causal = row_i >= col_i

    def mem_body(m, _):
        def chunk_body(i, carry):
            t0 = i * CHUNK
            q = q_scr[pl.ds(t0, CHUNK), :]
            w16 = w_scr[pl.ds(t0, CHUNK), :]
            lane = jax.lax.broadcasted_iota(jnp.int32, (CHUNK, 16), 1)
            wrow = jnp.sum(jnp.where(lane == m, w16, 0.0), axis=1,
                           keepdims=True)
            selc = wrow >= 0.0
            weight = jnp.maximum(wrow, 0.0)
            k = k_scr[pl.ds(t0, CHUNK), pl.ds(m * d, d)]
            k = jnp.where(selc, k, 0.0)
            v = v_scr[pl.ds(t0, CHUNK), pl.ds(m * d, d)]
            a = jax.lax.dot_general(q, k, (((1,), (1,)), ((), ())),
                                    preferred_element_type=F32)
            a = jnp.where(causal, a, 0.0)
            o = (jnp.dot(q, carry, preferred_element_type=F32)
                 + jnp.dot(a, v, preferred_element_type=F32))
            contrib = weight * o

            @pl.when(m == 0)
            def _init():
                out_ref[pl.ds(t0, CHUNK), :] = contrib

            @pl.when(m > 0)
            def _acc():
                out_ref[pl.ds(t0, CHUNK), :] += contrib

            return carry + jax.lax.dot_general(
                k, v, (((0,), (0,)), ((), ())), preferred_element_type=F32)

        jax.lax.fori_loop(0, L // CHUNK, chunk_body, m0)
        return 0

    jax.lax.fori_loop(0, NM + 1, mem_body, 0)


@functools.partial(jax.jit, static_argnames=("interpret",))
def kernel(X, M0, W_q, b_q, W_k, b_k, W_v, b_v, W_g, b_g, interpret=False):
    Mp1 = NM + 1
    x2 = X.reshape(L, B * D)
    wk_bf = W_k.astype(jnp.bfloat16)
    wv_bf = W_v.astype(jnp.bfloat16)
    bq2 = b_q.reshape(1, d)
    bk2 = b_k.reshape(1, Mp1 * d)
    bv2 = b_v.reshape(1, Mp1 * d)
    bg2 = b_g.reshape(NM, 1)

    out = pl.pallas_call(
        _mom_kernel,
        grid=(B,),
        in_specs=[
            pl.BlockSpec((L, D), lambda b: (0, b)),
            pl.BlockSpec((d, d), lambda b: (0, 0)),
            pl.BlockSpec((d, D), lambda b: (0, 0)),
            pl.BlockSpec((1, d), lambda b: (0, 0)),
            pl.BlockSpec((d * Mp1, D), lambda b: (0, 0)),
            pl.BlockSpec((1, Mp1 * d), lambda b: (0, 0)),
            pl.BlockSpec((d * Mp1, D), lambda b: (0, 0)),
            pl.BlockSpec((1, Mp1 * d), lambda b: (0, 0)),
            pl.BlockSpec((NM, D), lambda b: (0, 0)),
            pl.BlockSpec((NM, 1), lambda b: (0, 0)),
        ],
        out_specs=pl.BlockSpec((L, d), lambda b: (0, b)),
        out_shape=jax.ShapeDtypeStruct((L, B * d), F32),
        scratch_shapes=[
            pltpu.VMEM((L, d), F32),
            pltpu.VMEM((L, 16), F32),
            pltpu.VMEM((L, Mp1 * d), F32),
            pltpu.VMEM((L, Mp1 * d), F32),
        ],
        compiler_params=pltpu.CompilerParams(
            dimension_semantics=("parallel",),
        ),
        interpret=interpret,
    )(x2, M0, W_q, bq2, wk_bf, bk2, wv_bf, bv2, W_g, bg2)

    return out.reshape(L, B, d)
